# Initial kernel scaffold; baseline (speedup 1.0000x reference)
import dataclasses
import functools

import jax
import jax.numpy as jnp
from jax import lax
from jax.experimental import pallas as pl
from jax.experimental.pallas import tpu as pltpu
from jax.experimental.pallas import tpu_sc as plsc

N = 10000
HID = 128
R = 16
E = 320000
SEG = N * R

NC = 2
NS = 16
NW = NC * NS
LANES = 16

CHUNK = 80
EPW = E // NW
EPS = E // NS
SEG_PS = SEG // NW
NPS = N // NS

_f32 = jnp.float32
_i32 = jnp.int32


def _sc_compiler_params():
    cp = pltpu.CompilerParams()
    if "needs_layout_passes" in pltpu.CompilerParams.__dataclass_fields__:
        cp = dataclasses.replace(cp, needs_layout_passes=False)
    return cp


def _vector_mesh():
    return plsc.VectorSubcoreMesh(core_axis_name="c", subcore_axis_name="s")


def _inv_counts(comb, zeros_seg):
    @functools.partial(
        pl.kernel,
        out_type=jax.ShapeDtypeStruct((SEG,), _f32),
        mesh=_vector_mesh(),
        scratch_types=[
            pltpu.VMEM_SHARED((SEG,), _f32),
            pltpu.VMEM((CHUNK,), _i32),
            pltpu.VMEM((CHUNK,), _f32),
            pltpu.VMEM((SEG_PS,), _f32),
        ],
        compiler_params=_sc_compiler_params(),
    )
    def k(comb_hbm, zero_hbm, inv_hbm, cnt_sh, comb_v, ones_v, inv_v):
        c = lax.axis_index("c")
        s = lax.axis_index("s")

        @pl.when(s == 0)
        def _():
            pltpu.sync_copy(zero_hbm, cnt_sh)

        @pl.loop(0, CHUNK, step=LANES)
        def _(i):
            ones_v[pl.ds(i, LANES)] = jnp.full((LANES,), 1.0, _f32)

        plsc.subcore_barrier()

        base0 = s * EPS

        @pl.loop(0, EPS, step=CHUNK)
        def _(j):
            pltpu.sync_copy(comb_hbm.at[pl.ds(base0 + j, CHUNK)], comb_v)
            pltpu.sync_copy(ones_v, cnt_sh.at[comb_v], add=True)

        plsc.subcore_barrier()

        ibase = c * (SEG // NC) + s * SEG_PS
        pltpu.sync_copy(cnt_sh.at[pl.ds(ibase, SEG_PS)], inv_v)

        @pl.loop(0, SEG_PS, step=LANES)
        def _(i):
            v = inv_v[pl.ds(i, LANES)]
            inv_v[pl.ds(i, LANES)] = 1.0 / jnp.maximum(v, 1.0)

        pltpu.sync_copy(inv_v, inv_hbm.at[pl.ds(ibase, SEG_PS)])

    return k(comb, zeros_seg)


def _edge_norm(comb, inv):
    @functools.partial(
        pl.kernel,
        out_type=jax.ShapeDtypeStruct((E,), _f32),
        mesh=_vector_mesh(),
        scratch_types=[
            pltpu.VMEM((CHUNK,), _i32),
            pltpu.VMEM((CHUNK,), _f32),
            pltpu.SemaphoreType.DMA,
        ],
        compiler_params=_sc_compiler_params(),
    )
    def k(comb_hbm, inv_hbm, norm_hbm, comb_v, nv, sem):
        c = lax.axis_index("c")
        s = lax.axis_index("s")
        base = (s * NC + c) * EPW

        @pl.loop(0, EPW, step=CHUNK)
        def _(j):
            pltpu.sync_copy(comb_hbm.at[pl.ds(base + j, CHUNK)], comb_v)
            pltpu.async_copy(inv_hbm.at[comb_v], nv, sem).wait()
            pltpu.sync_copy(nv, norm_hbm.at[pl.ds(base + j, CHUNK)])

    return k(comb, inv)


def _scatter_layer(xr_flat, flat_idx, dst, norm, zeros_nd):
    @functools.partial(
        pl.kernel,
        out_type=jax.ShapeDtypeStruct((NC, N, HID), _f32),
        mesh=_vector_mesh(),
        scratch_types=[
            pltpu.VMEM_SHARED((N, HID), _f32),
            pltpu.VMEM((CHUNK,), _i32),
            pltpu.VMEM((CHUNK,), _i32),
            pltpu.VMEM((CHUNK,), _f32),
            pltpu.VMEM((CHUNK, HID), _f32),
            pltpu.SemaphoreType.DMA,
        ],
        compiler_params=_sc_compiler_params(),
    )
    def k(xr_hbm, fi_hbm, dst_hbm, nrm_hbm, zero_hbm, part_hbm,
          agg_sh, idx_v, dst_v, nv, rows_v, sem):
        c = lax.axis_index("c")
        s = lax.axis_index("s")

        @pl.when(s == 0)
        def _():
            pltpu.sync_copy(zero_hbm, agg_sh)

        plsc.subcore_barrier()

        base = (s * NC + c) * EPW

        @pl.loop(0, EPW, step=CHUNK)
        def _(j):
            pltpu.sync_copy(fi_hbm.at[pl.ds(base + j, CHUNK)], idx_v)
            pltpu.sync_copy(dst_hbm.at[pl.ds(base + j, CHUNK)], dst_v)
            pltpu.sync_copy(nrm_hbm.at[pl.ds(base + j, CHUNK)], nv)
            pltpu.async_copy(xr_hbm.at[idx_v], rows_v, sem).wait()

            @pl.loop(0, CHUNK)
            def _(i):
                splat = plsc.load_gather(nv, [jnp.broadcast_to(i, (LANES,))])
                for f in range(0, HID, LANES):
                    rows_v[i, pl.ds(f, LANES)] = rows_v[i, pl.ds(f, LANES)] * splat

            pltpu.sync_copy(rows_v, agg_sh.at[dst_v], add=True)

        plsc.subcore_barrier()

        rbase = s * NPS
        pltpu.sync_copy(agg_sh.at[pl.ds(rbase, NPS)],
                        part_hbm.at[c].at[pl.ds(rbase, NPS)])

    return k(xr_flat, flat_idx, dst, norm, zeros_nd)


_BN = 2000


def _xr_body(x_ref, w_ref, o_ref):
    o_ref[0] = jnp.dot(x_ref[...], w_ref[0], preferred_element_type=_f32)


def _per_relation_transform(x, W):
    return pl.pallas_call(
        _xr_body,
        grid=(N // _BN, R),
        in_specs=[
            pl.BlockSpec((_BN, HID), lambda n, r: (n, 0)),
            pl.BlockSpec((1, HID, HID), lambda n, r: (r, 0, 0)),
        ],
        out_specs=pl.BlockSpec((1, _BN, HID), lambda n, r: (r, n, 0)),
        out_shape=jax.ShapeDtypeStruct((R, N, HID), _f32),
    )(x, W)


def _combine_body(p_ref, x_ref, r_ref, b_ref, o_ref, *, relu):
    y = (p_ref[0] + p_ref[1]
         + jnp.dot(x_ref[...], r_ref[...], preferred_element_type=_f32)
         + b_ref[0])
    o_ref[...] = jnp.maximum(y, 0.0) if relu else y


def _combine(part, x, root, b, relu):
    return pl.pallas_call(
        functools.partial(_combine_body, relu=relu),
        grid=(N // _BN,),
        in_specs=[
            pl.BlockSpec((NC, _BN, HID), lambda n: (0, n, 0)),
            pl.BlockSpec((_BN, HID), lambda n: (n, 0)),
            pl.BlockSpec((HID, HID), lambda n: (0, 0)),
            pl.BlockSpec((1, HID), lambda n: (0, 0)),
        ],
        out_specs=pl.BlockSpec((_BN, HID), lambda n: (n, 0)),
        out_shape=jax.ShapeDtypeStruct((N, HID), _f32),
    )(part, x, root, b)


def kernel(edge_index, edge_type, node_emb, W1, root1, b1, W2, root2, b2):
    src = edge_index[0]
    dst = edge_index[1]
    rel = edge_type
    comb = dst * R + rel
    flat_idx = rel * N + src
    zeros_seg = jnp.zeros((SEG,), _f32)
    zeros_nd = jnp.zeros((N, HID), _f32)

    inv = _inv_counts(comb, zeros_seg)
    norm = _edge_norm(comb, inv)

    x = node_emb
    for W, root, b, relu in ((W1, root1, b1, True), (W2, root2, b2, False)):
        xr = _per_relation_transform(x, W).reshape(SEG, HID)
        part = _scatter_layer(xr, flat_idx, dst, norm, zeros_nd)
        x = _combine(part, x, root, b.reshape(1, HID), relu)
    return x


# speedup vs baseline: 5.4353x; 5.4353x over previous
import dataclasses
import functools

import jax
import jax.numpy as jnp
from jax import lax
from jax.experimental import pallas as pl
from jax.experimental.pallas import tpu as pltpu
from jax.experimental.pallas import tpu_sc as plsc

N = 10000
HID = 128
R = 16
E = 320000
SEG = N * R

NC = 2
NS = 16
NW = NC * NS
LANES = 16

CHUNK = 80
EPW = E // NW
EPS = E // NS
SEG_PS = SEG // NW
NPS = N // NS

_f32 = jnp.float32
_i32 = jnp.int32


def _sc_compiler_params():
    cp = pltpu.CompilerParams()
    if "needs_layout_passes" in pltpu.CompilerParams.__dataclass_fields__:
        cp = dataclasses.replace(cp, needs_layout_passes=False)
    return cp


def _vector_mesh():
    return plsc.VectorSubcoreMesh(core_axis_name="c", subcore_axis_name="s")


def _inv_counts(comb, zeros_seg):
    @functools.partial(
        pl.kernel,
        out_type=jax.ShapeDtypeStruct((SEG,), _f32),
        mesh=_vector_mesh(),
        scratch_types=[
            pltpu.VMEM_SHARED((SEG,), _f32),
            pltpu.VMEM((CHUNK,), _i32),
            pltpu.VMEM((CHUNK,), _f32),
            pltpu.VMEM((SEG_PS,), _f32),
        ],
        compiler_params=_sc_compiler_params(),
    )
    def k(comb_hbm, zero_hbm, inv_hbm, cnt_sh, comb_v, ones_v, inv_v):
        c = lax.axis_index("c")
        s = lax.axis_index("s")

        @pl.when(s == 0)
        def _():
            pltpu.sync_copy(zero_hbm, cnt_sh)

        @pl.loop(0, CHUNK, step=LANES)
        def _(i):
            ones_v[pl.ds(i, LANES)] = jnp.full((LANES,), 1.0, _f32)

        plsc.subcore_barrier()

        base0 = s * EPS

        @pl.loop(0, EPS, step=CHUNK)
        def _(j):
            pltpu.sync_copy(comb_hbm.at[pl.ds(base0 + j, CHUNK)], comb_v)
            pltpu.sync_copy(ones_v, cnt_sh.at[comb_v], add=True)

        plsc.subcore_barrier()

        ibase = c * (SEG // NC) + s * SEG_PS
        pltpu.sync_copy(cnt_sh.at[pl.ds(ibase, SEG_PS)], inv_v)

        @pl.loop(0, SEG_PS, step=LANES)
        def _(i):
            v = inv_v[pl.ds(i, LANES)]
            inv_v[pl.ds(i, LANES)] = 1.0 / jnp.maximum(v, 1.0)

        pltpu.sync_copy(inv_v, inv_hbm.at[pl.ds(ibase, SEG_PS)])

    return k(comb, zeros_seg)


def _edge_norm(comb, inv):
    @functools.partial(
        pl.kernel,
        out_type=jax.ShapeDtypeStruct((E,), _f32),
        mesh=_vector_mesh(),
        scratch_types=[
            pltpu.VMEM((CHUNK,), _i32),
            pltpu.VMEM((CHUNK,), _f32),
            pltpu.SemaphoreType.DMA,
        ],
        compiler_params=_sc_compiler_params(),
    )
    def k(comb_hbm, inv_hbm, norm_hbm, comb_v, nv, sem):
        c = lax.axis_index("c")
        s = lax.axis_index("s")
        base = (s * NC + c) * EPW

        @pl.loop(0, EPW, step=CHUNK)
        def _(j):
            pltpu.sync_copy(comb_hbm.at[pl.ds(base + j, CHUNK)], comb_v)
            pltpu.async_copy(inv_hbm.at[comb_v], nv, sem).wait()
            pltpu.sync_copy(nv, norm_hbm.at[pl.ds(base + j, CHUNK)])

    return k(comb, inv)


def _scatter_layer(xr_flat, flat_idx, dst, norm, zeros_nd):
    @functools.partial(
        pl.kernel,
        out_type=jax.ShapeDtypeStruct((NC, N, HID), _f32),
        mesh=_vector_mesh(),
        scratch_types=[
            pltpu.VMEM_SHARED((N, HID), _f32),
            pltpu.VMEM((CHUNK,), _i32),
            pltpu.VMEM((CHUNK,), _i32),
            pltpu.VMEM((CHUNK,), _f32),
            pltpu.VMEM((CHUNK, HID), _f32),
            pltpu.SemaphoreType.DMA,
        ],
        compiler_params=_sc_compiler_params(),
    )
    def k(xr_hbm, fi_hbm, dst_hbm, nrm_hbm, zero_hbm, part_hbm,
          agg_sh, idx_v, dst_v, nv, rows_v, sem):
        c = lax.axis_index("c")
        s = lax.axis_index("s")

        @pl.when(s == 0)
        def _():
            pltpu.sync_copy(zero_hbm, agg_sh)

        plsc.subcore_barrier()

        base = (s * NC + c) * EPW

        @pl.loop(0, EPW, step=CHUNK)
        def _(j):
            pltpu.sync_copy(fi_hbm.at[pl.ds(base + j, CHUNK)], idx_v)
            pltpu.sync_copy(dst_hbm.at[pl.ds(base + j, CHUNK)], dst_v)
            pltpu.sync_copy(nrm_hbm.at[pl.ds(base + j, CHUNK)], nv)
            pltpu.async_copy(xr_hbm.at[idx_v], rows_v, sem).wait()

            @pl.loop(0, CHUNK)
            def _(i):
                splat = plsc.load_gather(nv, [jnp.broadcast_to(i, (LANES,))])
                for f in range(0, HID, LANES):
                    rows_v[i, pl.ds(f, LANES)] = rows_v[i, pl.ds(f, LANES)] * splat

            pltpu.sync_copy(rows_v, agg_sh.at[dst_v], add=True)

        plsc.subcore_barrier()

        @pl.when(s < NS - 1)
        def _():
            pltpu.sync_copy(agg_sh.at[pl.ds(s * 624, 624)],
                            part_hbm.at[c].at[pl.ds(s * 624, 624)])

        @pl.when(s == NS - 1)
        def _():
            pltpu.sync_copy(agg_sh.at[pl.ds(624 * (NS - 1), N - 624 * (NS - 1))],
                            part_hbm.at[c].at[pl.ds(624 * (NS - 1), N - 624 * (NS - 1))])

    return k(xr_flat, flat_idx, dst, norm, zeros_nd)


_BN = 2000


def _xr_body(x_ref, w_ref, o_ref):
    o_ref[0] = jnp.dot(x_ref[...], w_ref[0], preferred_element_type=_f32)


def _per_relation_transform(x, W):
    return pl.pallas_call(
        _xr_body,
        grid=(N // _BN, R),
        in_specs=[
            pl.BlockSpec((_BN, HID), lambda n, r: (n, 0)),
            pl.BlockSpec((1, HID, HID), lambda n, r: (r, 0, 0)),
        ],
        out_specs=pl.BlockSpec((1, _BN, HID), lambda n, r: (r, n, 0)),
        out_shape=jax.ShapeDtypeStruct((R, N, HID), _f32),
    )(x, W)


def _combine_body(p_ref, x_ref, r_ref, b_ref, o_ref, *, relu):
    y = (p_ref[0] + p_ref[1]
         + jnp.dot(x_ref[...], r_ref[...], preferred_element_type=_f32)
         + b_ref[0])
    o_ref[...] = jnp.maximum(y, 0.0) if relu else y


def _combine(part, x, root, b, relu):
    return pl.pallas_call(
        functools.partial(_combine_body, relu=relu),
        grid=(N // _BN,),
        in_specs=[
            pl.BlockSpec((NC, _BN, HID), lambda n: (0, n, 0)),
            pl.BlockSpec((_BN, HID), lambda n: (n, 0)),
            pl.BlockSpec((HID, HID), lambda n: (0, 0)),
            pl.BlockSpec((1, HID), lambda n: (0, 0)),
        ],
        out_specs=pl.BlockSpec((_BN, HID), lambda n: (n, 0)),
        out_shape=jax.ShapeDtypeStruct((N, HID), _f32),
    )(part, x, root, b)


def kernel(edge_index, edge_type, node_emb, W1, root1, b1, W2, root2, b2):
    src = edge_index[0]
    dst = edge_index[1]
    rel = edge_type
    comb = dst * R + rel
    flat_idx = rel * N + src
    zeros_seg = jnp.zeros((SEG,), _f32)
    zeros_nd = jnp.zeros((N, HID), _f32)

    inv = _inv_counts(comb, zeros_seg)
    norm = _edge_norm(comb, inv)

    x = node_emb
    for W, root, b, relu in ((W1, root1, b1, True), (W2, root2, b2, False)):
        xr = _per_relation_transform(x, W).reshape(SEG, HID)
        part = _scatter_layer(xr, flat_idx, dst, norm, zeros_nd)
        x = _combine(part, x, root, b.reshape(1, HID), relu)
    return x


# speedup vs baseline: 13.0084x; 2.3933x over previous
import dataclasses
import functools

import jax
import jax.numpy as jnp
from jax import lax
from jax.experimental import pallas as pl
from jax.experimental.pallas import tpu as pltpu
from jax.experimental.pallas import tpu_sc as plsc

N = 10000
HID = 128
R = 16
E = 320000
SEG = N * R

NC = 2
NS = 16
NW = NC * NS
LANES = 16

CHUNK = 100
EPW = E // NW
NCH = EPW // CHUNK
EPS = E // NS
CCH = EPS // CHUNK
WIN = 10
SEG_PS = SEG // NW

_f32 = jnp.float32
_i32 = jnp.int32


def _sc_compiler_params():
    cp = pltpu.CompilerParams()
    if "needs_layout_passes" in pltpu.CompilerParams.__dataclass_fields__:
        cp = dataclasses.replace(cp, needs_layout_passes=False)
    return cp


def _vector_mesh():
    return plsc.VectorSubcoreMesh(core_axis_name="c", subcore_axis_name="s")


def _inv_counts(comb3, zeros_seg):
    @functools.partial(
        pl.kernel,
        out_type=jax.ShapeDtypeStruct((SEG,), _f32),
        mesh=_vector_mesh(),
        scratch_types=[
            pltpu.VMEM_SHARED((SEG,), _f32),
            pltpu.VMEM((CCH, CHUNK), _i32),
            pltpu.VMEM((CHUNK,), _f32),
            pltpu.VMEM((SEG_PS,), _f32),
            pltpu.SemaphoreType.DMA,
        ],
        compiler_params=_sc_compiler_params(),
    )
    def k(comb_hbm, zero_hbm, inv_hbm, cnt_sh, comb_v, ones_v, inv_v, sem):
        c = lax.axis_index("c")
        s = lax.axis_index("s")

        pltpu.sync_copy(comb_hbm.at[s], comb_v)

        @pl.when(s == 0)
        def _():
            pltpu.sync_copy(zero_hbm, cnt_sh)

        @pl.loop(0, CHUNK, step=LANES)
        def _(i):
            ones_v[pl.ds(i, LANES)] = jnp.full((LANES,), 1.0, _f32)

        plsc.subcore_barrier()

        @pl.loop(0, CCH, step=WIN)
        def _(j):
            for b in range(WIN):
                pltpu.async_copy(ones_v, cnt_sh.at[comb_v.at[j + b]], sem,
                                 add=True)
            for b in range(WIN):
                pltpu.make_async_copy(ones_v, cnt_sh.at[comb_v.at[j + b]],
                                      sem).wait()

        plsc.subcore_barrier()

        ibase = c * (SEG // NC) + s * SEG_PS
        pltpu.sync_copy(cnt_sh.at[pl.ds(ibase, SEG_PS)], inv_v)

        @pl.loop(0, SEG_PS, step=LANES)
        def _(i):
            v = inv_v[pl.ds(i, LANES)]
            inv_v[pl.ds(i, LANES)] = 1.0 / jnp.maximum(v, 1.0)

        pltpu.sync_copy(inv_v, inv_hbm.at[pl.ds(ibase, SEG_PS)])

    return k(comb3, zeros_seg)


def _edge_norm(comb4, inv):
    @functools.partial(
        pl.kernel,
        out_type=jax.ShapeDtypeStruct((NW, NCH, CHUNK), _f32),
        mesh=_vector_mesh(),
        scratch_types=[
            pltpu.VMEM((NCH, CHUNK), _i32),
            pltpu.VMEM((NCH, CHUNK), _f32),
            pltpu.SemaphoreType.DMA,
        ],
        compiler_params=_sc_compiler_params(),
    )
    def k(comb_hbm, inv_hbm, norm_hbm, comb_v, nv, sem):
        c = lax.axis_index("c")
        s = lax.axis_index("s")
        wid = s * NC + c
        pltpu.sync_copy(comb_hbm.at[wid], comb_v)

        @pl.loop(0, NCH, step=WIN)
        def _(j):
            for b in range(WIN):
                pltpu.async_copy(inv_hbm.at[comb_v.at[j + b]], nv.at[j + b],
                                 sem)
            for b in range(WIN):
                pltpu.make_async_copy(inv_hbm.at[comb_v.at[j + b]],
                                      nv.at[j + b], sem).wait()

        pltpu.sync_copy(nv, norm_hbm.at[wid])

    return k(comb4, inv)


def _scatter_layer(xr_flat, flat3, dst3, norm3, zeros_nd):
    @functools.partial(
        pl.kernel,
        out_type=jax.ShapeDtypeStruct((NC, N, HID), _f32),
        mesh=_vector_mesh(),
        scratch_types=[
            pltpu.VMEM_SHARED((N, HID), _f32),
            pltpu.VMEM((NCH, CHUNK), _i32),
            pltpu.VMEM((CHUNK,), _i32),
            pltpu.VMEM((CHUNK,), _i32),
            pltpu.VMEM((CHUNK,), _f32),
            pltpu.VMEM((CHUNK,), _f32),
            pltpu.VMEM((CHUNK, HID), _f32),
            pltpu.VMEM((CHUNK, HID), _f32),
            pltpu.SemaphoreType.DMA,
            pltpu.SemaphoreType.DMA,
        ],
        compiler_params=_sc_compiler_params(),
    )
    def k(xr_hbm, fi_hbm, dst_hbm, nrm_hbm, zero_hbm, part_hbm,
          agg_sh, flat_v, dst0, dst1, nrm0, nrm1, rows0, rows1, sem0, sem1):
        c = lax.axis_index("c")
        s = lax.axis_index("s")
        wid = s * NC + c

        pltpu.sync_copy(fi_hbm.at[wid], flat_v)

        @pl.when(s == 0)
        def _():
            pltpu.sync_copy(zero_hbm, agg_sh)

        plsc.subcore_barrier()

        rows = (rows0, rows1)
        dsts = (dst0, dst1)
        nrms = (nrm0, nrm1)
        sems = (sem0, sem1)
        for b in range(2):
            pltpu.async_copy(xr_hbm.at[flat_v.at[b]], rows[b], sems[b])
            pltpu.async_copy(dst_hbm.at[wid].at[b], dsts[b], sems[b])
            pltpu.async_copy(nrm_hbm.at[wid].at[b], nrms[b], sems[b])

        @pl.loop(0, NCH, step=2)
        def _(j):
            for b in range(2):
                jj = j + b
                rbuf = rows[b]
                sbuf = sems[b]
                pltpu.make_async_copy(xr_hbm.at[flat_v.at[jj]], rbuf,
                                      sbuf).wait()
                pltpu.make_async_copy(dst_hbm.at[wid].at[jj], dsts[b],
                                      sbuf).wait()
                pltpu.make_async_copy(nrm_hbm.at[wid].at[jj], nrms[b],
                                      sbuf).wait()

                @pl.loop(0, CHUNK)
                def _(i):
                    splat = plsc.load_gather(
                        nrms[b], [jnp.broadcast_to(i, (LANES,))])
                    for f in range(0, HID, LANES):
                        rbuf[i, pl.ds(f, LANES)] = (
                            rbuf[i, pl.ds(f, LANES)] * splat)

                pltpu.sync_copy(rbuf, agg_sh.at[dsts[b]], add=True)

                @pl.when(jj + 2 < NCH)
                def _():
                    pltpu.async_copy(xr_hbm.at[flat_v.at[jj + 2]], rbuf, sbuf)
                    pltpu.async_copy(dst_hbm.at[wid].at[jj + 2], dsts[b], sbuf)
                    pltpu.async_copy(nrm_hbm.at[wid].at[jj + 2], nrms[b], sbuf)

        plsc.subcore_barrier()

        @pl.when(s < NS - 1)
        def _():
            pltpu.sync_copy(agg_sh.at[pl.ds(s * 624, 624)],
                            part_hbm.at[c].at[pl.ds(s * 624, 624)])

        @pl.when(s == NS - 1)
        def _():
            pltpu.sync_copy(agg_sh.at[pl.ds(624 * (NS - 1), N - 624 * (NS - 1))],
                            part_hbm.at[c].at[pl.ds(624 * (NS - 1), N - 624 * (NS - 1))])

    return k(xr_flat, flat3, dst3, norm3, zeros_nd)


_BN = 2000


def _xr_body(x_ref, w_ref, o_ref):
    o_ref[0] = jnp.dot(x_ref[...], w_ref[0], preferred_element_type=_f32)


def _per_relation_transform(x, W):
    return pl.pallas_call(
        _xr_body,
        grid=(N // _BN, R),
        in_specs=[
            pl.BlockSpec((_BN, HID), lambda n, r: (n, 0)),
            pl.BlockSpec((1, HID, HID), lambda n, r: (r, 0, 0)),
        ],
        out_specs=pl.BlockSpec((1, _BN, HID), lambda n, r: (r, n, 0)),
        out_shape=jax.ShapeDtypeStruct((R, N, HID), _f32),
    )(x, W)


def _combine_body(p_ref, x_ref, r_ref, b_ref, o_ref, *, relu):
    y = (p_ref[0] + p_ref[1]
         + jnp.dot(x_ref[...], r_ref[...], preferred_element_type=_f32)
         + b_ref[0])
    o_ref[...] = jnp.maximum(y, 0.0) if relu else y


def _combine(part, x, root, b, relu):
    return pl.pallas_call(
        functools.partial(_combine_body, relu=relu),
        grid=(N // _BN,),
        in_specs=[
            pl.BlockSpec((NC, _BN, HID), lambda n: (0, n, 0)),
            pl.BlockSpec((_BN, HID), lambda n: (n, 0)),
            pl.BlockSpec((HID, HID), lambda n: (0, 0)),
            pl.BlockSpec((1, HID), lambda n: (0, 0)),
        ],
        out_specs=pl.BlockSpec((_BN, HID), lambda n: (n, 0)),
        out_shape=jax.ShapeDtypeStruct((N, HID), _f32),
    )(part, x, root, b)


def kernel(edge_index, edge_type, node_emb, W1, root1, b1, W2, root2, b2):
    src = edge_index[0]
    dst = edge_index[1]
    rel = edge_type
    comb = dst * R + rel
    flat_idx = rel * N + src
    comb3 = comb.reshape(NS, CCH, CHUNK)
    comb4 = comb.reshape(NW, NCH, CHUNK)
    flat3 = flat_idx.reshape(NW, NCH, CHUNK)
    dst3 = dst.reshape(NW, NCH, CHUNK)
    zeros_seg = jnp.zeros((SEG,), _f32)
    zeros_nd = jnp.zeros((N, HID), _f32)

    inv = _inv_counts(comb3, zeros_seg)
    norm3 = _edge_norm(comb4, inv)

    x = node_emb
    for W, root, b, relu in ((W1, root1, b1, True), (W2, root2, b2, False)):
        xr = _per_relation_transform(x, W).reshape(SEG, HID)
        part = _scatter_layer(xr, flat3, dst3, norm3, zeros_nd)
        x = _combine(part, x, root, b.reshape(1, HID), relu)
    return x


# speedup vs baseline: 14.4151x; 1.1081x over previous
import dataclasses
import functools

import jax
import jax.numpy as jnp
from jax import lax
from jax.experimental import pallas as pl
from jax.experimental.pallas import tpu as pltpu
from jax.experimental.pallas import tpu_sc as plsc

N = 10000
HID = 128
R = 16
E = 320000
SEG = N * R

NC = 2
NS = 16
NW = NC * NS
LANES = 16

CHUNK = 100
EPW = E // NW
NCH = EPW // CHUNK
EPS = E // NS
CCH = EPS // CHUNK
WIN = 10
SEG_PS = SEG // NW

_f32 = jnp.float32
_i32 = jnp.int32


def _sc_compiler_params():
    cp = pltpu.CompilerParams()
    if "needs_layout_passes" in pltpu.CompilerParams.__dataclass_fields__:
        cp = dataclasses.replace(cp, needs_layout_passes=False)
    return cp


def _vector_mesh():
    return plsc.VectorSubcoreMesh(core_axis_name="c", subcore_axis_name="s")


def _inv_counts(comb3, zeros_seg):
    @functools.partial(
        pl.kernel,
        out_type=jax.ShapeDtypeStruct((SEG,), _f32),
        mesh=_vector_mesh(),
        scratch_types=[
            pltpu.VMEM_SHARED((SEG,), _f32),
            pltpu.VMEM((CCH, CHUNK), _i32),
            pltpu.VMEM((CHUNK,), _f32),
            pltpu.VMEM((SEG_PS,), _f32),
            pltpu.SemaphoreType.DMA,
        ],
        compiler_params=_sc_compiler_params(),
    )
    def k(comb_hbm, zero_hbm, inv_hbm, cnt_sh, comb_v, ones_v, inv_v, sem):
        c = lax.axis_index("c")
        s = lax.axis_index("s")

        pltpu.sync_copy(comb_hbm.at[s], comb_v)

        @pl.when(s == 0)
        def _():
            pltpu.sync_copy(zero_hbm, cnt_sh)

        @pl.loop(0, CHUNK, step=LANES)
        def _(i):
            ones_v[pl.ds(i, LANES)] = jnp.full((LANES,), 1.0, _f32)

        plsc.subcore_barrier()

        @pl.loop(0, CCH, step=WIN)
        def _(j):
            for b in range(WIN):
                pltpu.async_copy(ones_v, cnt_sh.at[comb_v.at[j + b]], sem,
                                 add=True)
            for b in range(WIN):
                pltpu.make_async_copy(ones_v, cnt_sh.at[comb_v.at[j + b]],
                                      sem).wait()

        plsc.subcore_barrier()

        ibase = c * (SEG // NC) + s * SEG_PS
        pltpu.sync_copy(cnt_sh.at[pl.ds(ibase, SEG_PS)], inv_v)

        @pl.loop(0, SEG_PS, step=LANES)
        def _(i):
            v = inv_v[pl.ds(i, LANES)]
            inv_v[pl.ds(i, LANES)] = 1.0 / jnp.maximum(v, 1.0)

        pltpu.sync_copy(inv_v, inv_hbm.at[pl.ds(ibase, SEG_PS)])

    return k(comb3, zeros_seg)


def _edge_norm(comb4, inv):
    @functools.partial(
        pl.kernel,
        out_type=jax.ShapeDtypeStruct((NW, NCH, CHUNK), _f32),
        mesh=_vector_mesh(),
        scratch_types=[
            pltpu.VMEM((NCH, CHUNK), _i32),
            pltpu.VMEM((NCH, CHUNK), _f32),
            pltpu.SemaphoreType.DMA,
        ],
        compiler_params=_sc_compiler_params(),
    )
    def k(comb_hbm, inv_hbm, norm_hbm, comb_v, nv, sem):
        c = lax.axis_index("c")
        s = lax.axis_index("s")
        wid = s * NC + c
        pltpu.sync_copy(comb_hbm.at[wid], comb_v)

        @pl.loop(0, NCH, step=WIN)
        def _(j):
            for b in range(WIN):
                pltpu.async_copy(inv_hbm.at[comb_v.at[j + b]], nv.at[j + b],
                                 sem)
            for b in range(WIN):
                pltpu.make_async_copy(inv_hbm.at[comb_v.at[j + b]],
                                      nv.at[j + b], sem).wait()

        pltpu.sync_copy(nv, norm_hbm.at[wid])

    return k(comb4, inv)


def _scatter_layer(xr_flat, flat3, dst3, norm3, zeros_nd):
    @functools.partial(
        pl.kernel,
        out_type=jax.ShapeDtypeStruct((NC, N, HID), _f32),
        mesh=_vector_mesh(),
        scratch_types=[
            pltpu.VMEM_SHARED((N, HID), _f32),
            pltpu.VMEM((NCH, CHUNK), _i32),
            pltpu.VMEM((CHUNK,), _i32),
            pltpu.VMEM((CHUNK,), _i32),
            pltpu.VMEM((CHUNK,), _f32),
            pltpu.VMEM((CHUNK,), _f32),
            pltpu.VMEM((CHUNK, HID), _f32),
            pltpu.VMEM((CHUNK, HID), _f32),
            pltpu.SemaphoreType.DMA,
            pltpu.SemaphoreType.DMA,
            pltpu.SemaphoreType.DMA,
            pltpu.SemaphoreType.DMA,
        ],
        compiler_params=_sc_compiler_params(),
    )
    def k(xr_hbm, fi_hbm, dst_hbm, nrm_hbm, zero_hbm, part_hbm,
          agg_sh, flat_v, dst0, dst1, nrm0, nrm1, rows0, rows1,
          sem0, sem1, ssem0, ssem1):
        c = lax.axis_index("c")
        s = lax.axis_index("s")
        wid = s * NC + c

        pltpu.sync_copy(fi_hbm.at[wid], flat_v)

        @pl.when(s == 0)
        def _():
            pltpu.sync_copy(zero_hbm, agg_sh)

        plsc.subcore_barrier()

        rows = (rows0, rows1)
        dsts = (dst0, dst1)
        nrms = (nrm0, nrm1)
        sems = (sem0, sem1)
        ssems = (ssem0, ssem1)
        for b in range(2):
            pltpu.async_copy(xr_hbm.at[flat_v.at[b]], rows[b], sems[b])
            pltpu.async_copy(dst_hbm.at[wid].at[b], dsts[b], sems[b])
            pltpu.async_copy(nrm_hbm.at[wid].at[b], nrms[b], sems[b])

        @pl.loop(0, NCH, step=2)
        def _(j):
            for b in range(2):
                jj = j + b
                rbuf = rows[b]
                sbuf = sems[b]
                pltpu.make_async_copy(xr_hbm.at[flat_v.at[jj]], rbuf,
                                      sbuf).wait()
                pltpu.make_async_copy(dst_hbm.at[wid].at[jj], dsts[b],
                                      sbuf).wait()
                pltpu.make_async_copy(nrm_hbm.at[wid].at[jj], nrms[b],
                                      sbuf).wait()

                @pl.loop(0, CHUNK, step=2)
                def _(i):
                    sp0 = plsc.load_gather(
                        nrms[b], [jnp.broadcast_to(i, (LANES,))])
                    sp1 = plsc.load_gather(
                        nrms[b], [jnp.broadcast_to(i + 1, (LANES,))])
                    for f in range(0, HID, LANES):
                        rbuf[i, pl.ds(f, LANES)] = (
                            rbuf[i, pl.ds(f, LANES)] * sp0)
                        rbuf[i + 1, pl.ds(f, LANES)] = (
                            rbuf[i + 1, pl.ds(f, LANES)] * sp1)

                pltpu.async_copy(rbuf, agg_sh.at[dsts[b]], ssems[b], add=True)

                @pl.when(jj + 2 < NCH)
                def _():
                    pltpu.make_async_copy(rbuf, agg_sh.at[dsts[b]],
                                          ssems[b]).wait()
                    pltpu.async_copy(xr_hbm.at[flat_v.at[jj + 2]], rbuf, sbuf)
                    pltpu.async_copy(dst_hbm.at[wid].at[jj + 2], dsts[b], sbuf)
                    pltpu.async_copy(nrm_hbm.at[wid].at[jj + 2], nrms[b], sbuf)

        for b in range(2):
            pltpu.make_async_copy(rows[b], agg_sh.at[dsts[b]], ssems[b]).wait()

        plsc.subcore_barrier()

        @pl.when(s < NS - 1)
        def _():
            pltpu.sync_copy(agg_sh.at[pl.ds(s * 624, 624)],
                            part_hbm.at[c].at[pl.ds(s * 624, 624)])

        @pl.when(s == NS - 1)
        def _():
            pltpu.sync_copy(agg_sh.at[pl.ds(624 * (NS - 1), N - 624 * (NS - 1))],
                            part_hbm.at[c].at[pl.ds(624 * (NS - 1), N - 624 * (NS - 1))])

    return k(xr_flat, flat3, dst3, norm3, zeros_nd)


_BN = 2000


def _xr_body(x_ref, w_ref, o_ref):
    o_ref[0] = jnp.dot(x_ref[...], w_ref[0], preferred_element_type=_f32)


def _per_relation_transform(x, W):
    return pl.pallas_call(
        _xr_body,
        grid=(N // _BN, R),
        in_specs=[
            pl.BlockSpec((_BN, HID), lambda n, r: (n, 0)),
            pl.BlockSpec((1, HID, HID), lambda n, r: (r, 0, 0)),
        ],
        out_specs=pl.BlockSpec((1, _BN, HID), lambda n, r: (r, n, 0)),
        out_shape=jax.ShapeDtypeStruct((R, N, HID), _f32),
    )(x, W)


def _combine_body(p_ref, x_ref, r_ref, b_ref, o_ref, *, relu):
    y = (p_ref[0] + p_ref[1]
         + jnp.dot(x_ref[...], r_ref[...], preferred_element_type=_f32)
         + b_ref[0])
    o_ref[...] = jnp.maximum(y, 0.0) if relu else y


def _combine(part, x, root, b, relu):
    return pl.pallas_call(
        functools.partial(_combine_body, relu=relu),
        grid=(N // _BN,),
        in_specs=[
            pl.BlockSpec((NC, _BN, HID), lambda n: (0, n, 0)),
            pl.BlockSpec((_BN, HID), lambda n: (n, 0)),
            pl.BlockSpec((HID, HID), lambda n: (0, 0)),
            pl.BlockSpec((1, HID), lambda n: (0, 0)),
        ],
        out_specs=pl.BlockSpec((_BN, HID), lambda n: (n, 0)),
        out_shape=jax.ShapeDtypeStruct((N, HID), _f32),
    )(part, x, root, b)


def kernel(edge_index, edge_type, node_emb, W1, root1, b1, W2, root2, b2):
    src = edge_index[0]
    dst = edge_index[1]
    rel = edge_type
    comb = dst * R + rel
    flat_idx = rel * N + src
    comb3 = comb.reshape(NS, CCH, CHUNK)
    comb4 = comb.reshape(NW, NCH, CHUNK)
    flat3 = flat_idx.reshape(NW, NCH, CHUNK)
    dst3 = dst.reshape(NW, NCH, CHUNK)
    zeros_seg = jnp.zeros((SEG,), _f32)
    zeros_nd = jnp.zeros((N, HID), _f32)

    inv = _inv_counts(comb3, zeros_seg)
    norm3 = _edge_norm(comb4, inv)

    x = node_emb
    for W, root, b, relu in ((W1, root1, b1, True), (W2, root2, b2, False)):
        xr = _per_relation_transform(x, W).reshape(SEG, HID)
        part = _scatter_layer(xr, flat3, dst3, norm3, zeros_nd)
        x = _combine(part, x, root, b.reshape(1, HID), relu)
    return x


# speedup vs baseline: 14.8346x; 1.0291x over previous
import dataclasses
import functools

import jax
import jax.numpy as jnp
from jax import lax
from jax.experimental import pallas as pl
from jax.experimental.pallas import tpu as pltpu
from jax.experimental.pallas import tpu_sc as plsc

N = 10000
HID = 128
R = 16
E = 320000
SEG = N * R

NC = 2
NS = 16
NW = NC * NS
LANES = 16

CHUNK = 100
EPW = E // NW
NCH = EPW // CHUNK
EPS = E // NS
CCH = EPS // CHUNK
WIN = 10
SEG_PS = SEG // NW

_f32 = jnp.float32
_i32 = jnp.int32


def _sc_compiler_params():
    cp = pltpu.CompilerParams()
    if "needs_layout_passes" in pltpu.CompilerParams.__dataclass_fields__:
        cp = dataclasses.replace(cp, needs_layout_passes=False)
    return cp


def _vector_mesh():
    return plsc.VectorSubcoreMesh(core_axis_name="c", subcore_axis_name="s")


def _edge_norm(comb3, comb4, zeros_seg):
    @functools.partial(
        pl.kernel,
        out_type=(jax.ShapeDtypeStruct((NC * SEG,), _f32),
                  jax.ShapeDtypeStruct((NW, NCH, CHUNK), _f32)),
        mesh=_vector_mesh(),
        scratch_types=[
            pltpu.VMEM_SHARED((SEG,), _f32),
            pltpu.VMEM((CCH, CHUNK), _i32),
            pltpu.VMEM((CHUNK,), _f32),
            pltpu.VMEM((SEG - 9984 * (NS - 1),), _f32),
            pltpu.VMEM((NCH, CHUNK), _f32),
            pltpu.SemaphoreType.DMA,
        ],
        compiler_params=_sc_compiler_params(),
    )
    def k(comb3_hbm, comb4_hbm, zero_hbm, inv_hbm, norm_hbm,
          cnt_sh, comb_v, ones_v, inv_v, nv, sem):
        c = lax.axis_index("c")
        s = lax.axis_index("s")
        wid = s * NC + c

        pltpu.sync_copy(comb3_hbm.at[s], comb_v)

        @pl.when(s == 0)
        def _():
            pltpu.sync_copy(zero_hbm, cnt_sh)

        @pl.loop(0, CHUNK, step=LANES)
        def _(i):
            ones_v[pl.ds(i, LANES)] = jnp.full((LANES,), 1.0, _f32)

        plsc.subcore_barrier()

        @pl.loop(0, CCH, step=WIN)
        def _(j):
            for b in range(WIN):
                pltpu.async_copy(ones_v, cnt_sh.at[comb_v.at[j + b]], sem,
                                 add=True)
            for b in range(WIN):
                pltpu.make_async_copy(ones_v, cnt_sh.at[comb_v.at[j + b]],
                                      sem).wait()

        plsc.subcore_barrier()

        def _inv_stripe(ibase, size):
            pltpu.sync_copy(cnt_sh.at[pl.ds(ibase, size)],
                            inv_v.at[pl.ds(0, size)])

            @pl.loop(0, size, step=LANES)
            def _(i):
                v = inv_v[pl.ds(i, LANES)]
                inv_v[pl.ds(i, LANES)] = 1.0 / jnp.maximum(v, 1.0)

            pltpu.sync_copy(inv_v.at[pl.ds(0, size)],
                            inv_hbm.at[pl.ds(c * SEG + ibase, size)])

        @pl.when(s < NS - 1)
        def _():
            _inv_stripe(s * 9984, 9984)

        @pl.when(s == NS - 1)
        def _():
            _inv_stripe(9984 * (NS - 1), SEG - 9984 * (NS - 1))

        plsc.subcore_barrier()

        pltpu.sync_copy(comb4_hbm.at[wid], comb_v.at[pl.ds(0, NCH)])
        off = jnp.broadcast_to(c * SEG, (LANES,)).astype(_i32)

        @pl.loop(0, NCH)
        def _(j):
            @pl.loop(0, CHUNK, step=LANES)
            def _(t):
                comb_v[j, pl.ds(t, LANES)] = comb_v[j, pl.ds(t, LANES)] + off

        @pl.loop(0, NCH, step=WIN)
        def _(j):
            for b in range(WIN):
                pltpu.async_copy(inv_hbm.at[comb_v.at[j + b]],
                                 nv.at[j + b], sem)
            for b in range(WIN):
                pltpu.make_async_copy(inv_hbm.at[comb_v.at[j + b]],
                                      nv.at[j + b], sem).wait()

        pltpu.sync_copy(nv, norm_hbm.at[wid])

    return k(comb3, comb4, zeros_seg)[1]


def _scatter_layer(xr_flat, flat3, dst3, norm3, zeros_nd):
    @functools.partial(
        pl.kernel,
        out_type=jax.ShapeDtypeStruct((NC, N, HID), _f32),
        mesh=_vector_mesh(),
        scratch_types=[
            pltpu.VMEM_SHARED((N, HID), _f32),
            pltpu.VMEM((NCH, CHUNK), _i32),
            pltpu.VMEM((CHUNK,), _i32),
            pltpu.VMEM((CHUNK,), _i32),
            pltpu.VMEM((CHUNK,), _f32),
            pltpu.VMEM((CHUNK,), _f32),
            pltpu.VMEM((CHUNK, HID), _f32),
            pltpu.VMEM((CHUNK, HID), _f32),
            pltpu.SemaphoreType.DMA,
            pltpu.SemaphoreType.DMA,
            pltpu.SemaphoreType.DMA,
            pltpu.SemaphoreType.DMA,
        ],
        compiler_params=_sc_compiler_params(),
    )
    def k(xr_hbm, fi_hbm, dst_hbm, nrm_hbm, zero_hbm, part_hbm,
          agg_sh, flat_v, dst0, dst1, nrm0, nrm1, rows0, rows1,
          sem0, sem1, ssem0, ssem1):
        c = lax.axis_index("c")
        s = lax.axis_index("s")
        wid = s * NC + c

        pltpu.sync_copy(fi_hbm.at[wid], flat_v)

        @pl.when(s == 0)
        def _():
            pltpu.sync_copy(zero_hbm, agg_sh)

        plsc.subcore_barrier()

        rows = (rows0, rows1)
        dsts = (dst0, dst1)
        nrms = (nrm0, nrm1)
        sems = (sem0, sem1)
        ssems = (ssem0, ssem1)
        for b in range(2):
            pltpu.async_copy(xr_hbm.at[flat_v.at[b]], rows[b], sems[b])
            pltpu.async_copy(dst_hbm.at[wid].at[b], dsts[b], sems[b])
            pltpu.async_copy(nrm_hbm.at[wid].at[b], nrms[b], sems[b])

        @pl.loop(0, NCH, step=2)
        def _(j):
            for b in range(2):
                jj = j + b
                rbuf = rows[b]
                sbuf = sems[b]
                pltpu.make_async_copy(xr_hbm.at[flat_v.at[jj]], rbuf,
                                      sbuf).wait()
                pltpu.make_async_copy(dst_hbm.at[wid].at[jj], dsts[b],
                                      sbuf).wait()
                pltpu.make_async_copy(nrm_hbm.at[wid].at[jj], nrms[b],
                                      sbuf).wait()

                @pl.loop(0, CHUNK, step=2)
                def _(i):
                    sp0 = plsc.load_gather(
                        nrms[b], [jnp.broadcast_to(i, (LANES,))])
                    sp1 = plsc.load_gather(
                        nrms[b], [jnp.broadcast_to(i + 1, (LANES,))])
                    for f in range(0, HID, LANES):
                        rbuf[i, pl.ds(f, LANES)] = (
                            rbuf[i, pl.ds(f, LANES)] * sp0)
                        rbuf[i + 1, pl.ds(f, LANES)] = (
                            rbuf[i + 1, pl.ds(f, LANES)] * sp1)

                pltpu.async_copy(rbuf, agg_sh.at[dsts[b]], ssems[b], add=True)

                @pl.when(jj + 2 < NCH)
                def _():
                    pltpu.make_async_copy(rbuf, agg_sh.at[dsts[b]],
                                          ssems[b]).wait()
                    pltpu.async_copy(xr_hbm.at[flat_v.at[jj + 2]], rbuf, sbuf)
                    pltpu.async_copy(dst_hbm.at[wid].at[jj + 2], dsts[b], sbuf)
                    pltpu.async_copy(nrm_hbm.at[wid].at[jj + 2], nrms[b], sbuf)

        for b in range(2):
            pltpu.make_async_copy(rows[b], agg_sh.at[dsts[b]], ssems[b]).wait()

        plsc.subcore_barrier()

        @pl.when(s < NS - 1)
        def _():
            pltpu.sync_copy(agg_sh.at[pl.ds(s * 624, 624)],
                            part_hbm.at[c].at[pl.ds(s * 624, 624)])

        @pl.when(s == NS - 1)
        def _():
            pltpu.sync_copy(agg_sh.at[pl.ds(624 * (NS - 1), N - 624 * (NS - 1))],
                            part_hbm.at[c].at[pl.ds(624 * (NS - 1), N - 624 * (NS - 1))])

    return k(xr_flat, flat3, dst3, norm3, zeros_nd)


_BN = 2000


def _xr_body(x_ref, w_ref, o_ref):
    o_ref[0] = jnp.dot(x_ref[...], w_ref[0], preferred_element_type=_f32)


def _per_relation_transform(x, W):
    return pl.pallas_call(
        _xr_body,
        grid=(N // _BN, R),
        in_specs=[
            pl.BlockSpec((_BN, HID), lambda n, r: (n, 0)),
            pl.BlockSpec((1, HID, HID), lambda n, r: (r, 0, 0)),
        ],
        out_specs=pl.BlockSpec((1, _BN, HID), lambda n, r: (r, n, 0)),
        out_shape=jax.ShapeDtypeStruct((R, N, HID), _f32),
    )(x, W)


def _combine_transform_body(p_ref, x_ref, r_ref, b_ref, w_ref,
                            x1_ref, xr_ref, x1s):
    @pl.when(pl.program_id(1) == 0)
    def _():
        y = (p_ref[0] + p_ref[1]
             + jnp.dot(x_ref[...], r_ref[...], preferred_element_type=_f32)
             + b_ref[0])
        y = jnp.maximum(y, 0.0)
        x1s[...] = y
        x1_ref[...] = y

    xr_ref[0] = jnp.dot(x1s[...], w_ref[0], preferred_element_type=_f32)


def _combine_transform(part, x, root, b, Wnext):
    return pl.pallas_call(
        _combine_transform_body,
        grid=(N // _BN, R),
        in_specs=[
            pl.BlockSpec((NC, _BN, HID), lambda n, r: (0, n, 0)),
            pl.BlockSpec((_BN, HID), lambda n, r: (n, 0)),
            pl.BlockSpec((HID, HID), lambda n, r: (0, 0)),
            pl.BlockSpec((1, HID), lambda n, r: (0, 0)),
            pl.BlockSpec((1, HID, HID), lambda n, r: (r, 0, 0)),
        ],
        out_specs=[
            pl.BlockSpec((_BN, HID), lambda n, r: (n, 0)),
            pl.BlockSpec((1, _BN, HID), lambda n, r: (r, n, 0)),
        ],
        out_shape=[
            jax.ShapeDtypeStruct((N, HID), _f32),
            jax.ShapeDtypeStruct((R, N, HID), _f32),
        ],
        scratch_shapes=[pltpu.VMEM((_BN, HID), _f32)],
    )(part, x, root, b, Wnext)


def _combine_body(p_ref, x_ref, r_ref, b_ref, o_ref, *, relu):
    y = (p_ref[0] + p_ref[1]
         + jnp.dot(x_ref[...], r_ref[...], preferred_element_type=_f32)
         + b_ref[0])
    o_ref[...] = jnp.maximum(y, 0.0) if relu else y


def _combine(part, x, root, b, relu):
    return pl.pallas_call(
        functools.partial(_combine_body, relu=relu),
        grid=(N // _BN,),
        in_specs=[
            pl.BlockSpec((NC, _BN, HID), lambda n: (0, n, 0)),
            pl.BlockSpec((_BN, HID), lambda n: (n, 0)),
            pl.BlockSpec((HID, HID), lambda n: (0, 0)),
            pl.BlockSpec((1, HID), lambda n: (0, 0)),
        ],
        out_specs=pl.BlockSpec((_BN, HID), lambda n: (n, 0)),
        out_shape=jax.ShapeDtypeStruct((N, HID), _f32),
    )(part, x, root, b)


def kernel(edge_index, edge_type, node_emb, W1, root1, b1, W2, root2, b2):
    src = edge_index[0]
    dst = edge_index[1]
    rel = edge_type
    comb = dst * R + rel
    flat_idx = rel * N + src
    comb3 = comb.reshape(NS, CCH, CHUNK)
    comb4 = comb.reshape(NW, NCH, CHUNK)
    flat3 = flat_idx.reshape(NW, NCH, CHUNK)
    dst3 = dst.reshape(NW, NCH, CHUNK)
    zeros_seg = jnp.zeros((SEG,), _f32)
    zeros_nd = jnp.zeros((N, HID), _f32)

    norm3 = _edge_norm(comb3, comb4, zeros_seg)

    xr1 = _per_relation_transform(node_emb, W1).reshape(SEG, HID)
    part1 = _scatter_layer(xr1, flat3, dst3, norm3, zeros_nd)
    x1, xr2 = _combine_transform(part1, node_emb, root1,
                                 b1.reshape(1, HID), W2)
    part2 = _scatter_layer(xr2.reshape(SEG, HID), flat3, dst3, norm3,
                           zeros_nd)
    return _combine(part2, x1, root2, b2.reshape(1, HID), relu=False)


# speedup vs baseline: 17.0634x; 1.1502x over previous
import dataclasses
import functools

import jax
import jax.numpy as jnp
from jax import lax
from jax.experimental import pallas as pl
from jax.experimental.pallas import tpu as pltpu
from jax.experimental.pallas import tpu_sc as plsc

N = 10000
HID = 128
R = 16
E = 320000
SEG = N * R

NC = 2
NS = 16
NW = NC * NS
LANES = 16

CHUNK = 100
EPW = E // NW
NCH = EPW // CHUNK
EPS = E // NS
CCH = EPS // CHUNK
WIN = 10
SEG_PS = SEG // NW

_f32 = jnp.float32
_i32 = jnp.int32


def _sc_compiler_params():
    cp = pltpu.CompilerParams()
    if "needs_layout_passes" in pltpu.CompilerParams.__dataclass_fields__:
        cp = dataclasses.replace(cp, needs_layout_passes=False)
    return cp


def _vector_mesh():
    return plsc.VectorSubcoreMesh(core_axis_name="c", subcore_axis_name="s")


def _edge_norm(comb3, comb4, zeros_seg):
    @functools.partial(
        pl.kernel,
        out_type=(jax.ShapeDtypeStruct((NC * SEG,), _f32),
                  jax.ShapeDtypeStruct((NW, NCH, CHUNK), _f32)),
        mesh=_vector_mesh(),
        scratch_types=[
            pltpu.VMEM_SHARED((SEG,), _f32),
            pltpu.VMEM((CCH, CHUNK), _i32),
            pltpu.VMEM((CHUNK,), _f32),
            pltpu.VMEM((SEG - 9984 * (NS - 1),), _f32),
            pltpu.VMEM((NCH, CHUNK), _f32),
            pltpu.SemaphoreType.DMA,
        ],
        compiler_params=_sc_compiler_params(),
    )
    def k(comb3_hbm, comb4_hbm, zero_hbm, inv_hbm, norm_hbm,
          cnt_sh, comb_v, ones_v, inv_v, nv, sem):
        c = lax.axis_index("c")
        s = lax.axis_index("s")
        wid = s * NC + c

        pltpu.sync_copy(comb3_hbm.at[s], comb_v)

        @pl.when(s == 0)
        def _():
            pltpu.sync_copy(zero_hbm, cnt_sh)

        @pl.loop(0, CHUNK, step=LANES)
        def _(i):
            ones_v[pl.ds(i, LANES)] = jnp.full((LANES,), 1.0, _f32)

        plsc.subcore_barrier()

        @pl.loop(0, CCH, step=WIN)
        def _(j):
            for b in range(WIN):
                pltpu.async_copy(ones_v, cnt_sh.at[comb_v.at[j + b]], sem,
                                 add=True)
            for b in range(WIN):
                pltpu.make_async_copy(ones_v, cnt_sh.at[comb_v.at[j + b]],
                                      sem).wait()

        plsc.subcore_barrier()

        def _inv_stripe(ibase, size):
            pltpu.sync_copy(cnt_sh.at[pl.ds(ibase, size)],
                            inv_v.at[pl.ds(0, size)])

            @pl.loop(0, size, step=LANES)
            def _(i):
                v = inv_v[pl.ds(i, LANES)]
                inv_v[pl.ds(i, LANES)] = 1.0 / jnp.maximum(v, 1.0)

            pltpu.sync_copy(inv_v.at[pl.ds(0, size)],
                            inv_hbm.at[pl.ds(c * SEG + ibase, size)])

        @pl.when(s < NS - 1)
        def _():
            _inv_stripe(s * 9984, 9984)

        @pl.when(s == NS - 1)
        def _():
            _inv_stripe(9984 * (NS - 1), SEG - 9984 * (NS - 1))

        plsc.subcore_barrier()

        pltpu.sync_copy(comb4_hbm.at[wid], comb_v.at[pl.ds(0, NCH)])
        off = jnp.broadcast_to(c * SEG, (LANES,)).astype(_i32)

        @pl.loop(0, NCH)
        def _(j):
            @pl.loop(0, CHUNK, step=LANES)
            def _(t):
                comb_v[j, pl.ds(t, LANES)] = comb_v[j, pl.ds(t, LANES)] + off

        @pl.loop(0, NCH, step=WIN)
        def _(j):
            for b in range(WIN):
                pltpu.async_copy(inv_hbm.at[comb_v.at[j + b]],
                                 nv.at[j + b], sem)
            for b in range(WIN):
                pltpu.make_async_copy(inv_hbm.at[comb_v.at[j + b]],
                                      nv.at[j + b], sem).wait()

        pltpu.sync_copy(nv, norm_hbm.at[wid])

    return k(comb3, comb4, zeros_seg)[1]


def _scatter_layer(xr_flat, flat3, dst3, norm3, zeros_nd):
    @functools.partial(
        pl.kernel,
        out_type=jax.ShapeDtypeStruct((NC, N, HID), _f32),
        mesh=_vector_mesh(),
        scratch_types=[
            pltpu.VMEM_SHARED((N, HID), _f32),
            pltpu.VMEM((NCH, CHUNK), _i32),
            pltpu.VMEM((CHUNK,), _i32),
            pltpu.VMEM((CHUNK,), _i32),
            pltpu.VMEM((CHUNK,), _f32),
            pltpu.VMEM((CHUNK,), _f32),
            pltpu.VMEM((CHUNK, HID), _f32),
            pltpu.VMEM((CHUNK, HID), _f32),
            pltpu.SemaphoreType.DMA,
            pltpu.SemaphoreType.DMA,
            pltpu.SemaphoreType.DMA,
            pltpu.SemaphoreType.DMA,
        ],
        compiler_params=_sc_compiler_params(),
    )
    def k(xr_hbm, fi_hbm, dst_hbm, nrm_hbm, zero_hbm, part_hbm,
          agg_sh, flat_v, dst0, dst1, nrm0, nrm1, rows0, rows1,
          sem0, sem1, ssem0, ssem1):
        c = lax.axis_index("c")
        s = lax.axis_index("s")
        wid = s * NC + c

        pltpu.sync_copy(fi_hbm.at[wid], flat_v)

        @pl.when(s == 0)
        def _():
            pltpu.sync_copy(zero_hbm, agg_sh)

        plsc.subcore_barrier()

        rows = (rows0, rows1)
        dsts = (dst0, dst1)
        nrms = (nrm0, nrm1)
        sems = (sem0, sem1)
        ssems = (ssem0, ssem1)
        for b in range(2):
            pltpu.async_copy(xr_hbm.at[flat_v.at[b]], rows[b], sems[b])
            pltpu.async_copy(dst_hbm.at[wid].at[b], dsts[b], sems[b])
            pltpu.async_copy(nrm_hbm.at[wid].at[b], nrms[b], sems[b])

        @pl.loop(0, NCH, step=2)
        def _(j):
            for b in range(2):
                jj = j + b
                rbuf = rows[b]
                sbuf = sems[b]
                pltpu.make_async_copy(xr_hbm.at[flat_v.at[jj]], rbuf,
                                      sbuf).wait()
                pltpu.make_async_copy(dst_hbm.at[wid].at[jj], dsts[b],
                                      sbuf).wait()
                pltpu.make_async_copy(nrm_hbm.at[wid].at[jj], nrms[b],
                                      sbuf).wait()

                @pl.loop(0, CHUNK, step=2)
                def _(i):
                    sp0 = plsc.load_gather(
                        nrms[b], [jnp.broadcast_to(i, (LANES,))])
                    sp1 = plsc.load_gather(
                        nrms[b], [jnp.broadcast_to(i + 1, (LANES,))])
                    for f in range(0, HID, LANES):
                        rbuf[i, pl.ds(f, LANES)] = (
                            rbuf[i, pl.ds(f, LANES)] * sp0)
                        rbuf[i + 1, pl.ds(f, LANES)] = (
                            rbuf[i + 1, pl.ds(f, LANES)] * sp1)

                pltpu.async_copy(rbuf, agg_sh.at[dsts[b]], ssems[b], add=True)

                @pl.when(jj + 2 < NCH)
                def _():
                    pltpu.make_async_copy(rbuf, agg_sh.at[dsts[b]],
                                          ssems[b]).wait()
                    pltpu.async_copy(xr_hbm.at[flat_v.at[jj + 2]], rbuf, sbuf)
                    pltpu.async_copy(dst_hbm.at[wid].at[jj + 2], dsts[b], sbuf)
                    pltpu.async_copy(nrm_hbm.at[wid].at[jj + 2], nrms[b], sbuf)

        for b in range(2):
            pltpu.make_async_copy(rows[b], agg_sh.at[dsts[b]], ssems[b]).wait()

        plsc.subcore_barrier()

        @pl.when(s < NS - 1)
        def _():
            pltpu.sync_copy(agg_sh.at[pl.ds(s * 624, 624)],
                            part_hbm.at[c].at[pl.ds(s * 624, 624)])

        @pl.when(s == NS - 1)
        def _():
            pltpu.sync_copy(agg_sh.at[pl.ds(624 * (NS - 1), N - 624 * (NS - 1))],
                            part_hbm.at[c].at[pl.ds(624 * (NS - 1), N - 624 * (NS - 1))])

    return k(xr_flat, flat3, dst3, norm3, zeros_nd)


_BN = 2000


def _xr_body(x_ref, w_ref, o_ref):
    o_ref[0] = jnp.dot(x_ref[...], w_ref[0], preferred_element_type=_f32)


def _per_relation_transform(x, W):
    return pl.pallas_call(
        _xr_body,
        grid=(R,),
        in_specs=[
            pl.BlockSpec((N, HID), lambda r: (0, 0)),
            pl.BlockSpec((1, HID, HID), lambda r: (r, 0, 0)),
        ],
        out_specs=pl.BlockSpec((1, N, HID), lambda r: (r, 0, 0)),
        out_shape=jax.ShapeDtypeStruct((R, N, HID), _f32),
    )(x, W)


def _combine_transform_body(p_ref, x_ref, r_ref, b_ref, w_ref,
                            x1_ref, xr_ref, x1s):
    @pl.when(pl.program_id(0) == 0)
    def _():
        y = (p_ref[0] + p_ref[1]
             + jnp.dot(x_ref[...], r_ref[...], preferred_element_type=_f32)
             + b_ref[0])
        y = jnp.maximum(y, 0.0)
        x1s[...] = y
        x1_ref[...] = y

    xr_ref[0] = jnp.dot(x1s[...], w_ref[0], preferred_element_type=_f32)


def _combine_transform(part, x, root, b, Wnext):
    return pl.pallas_call(
        _combine_transform_body,
        grid=(R,),
        in_specs=[
            pl.BlockSpec((NC, N, HID), lambda r: (0, 0, 0)),
            pl.BlockSpec((N, HID), lambda r: (0, 0)),
            pl.BlockSpec((HID, HID), lambda r: (0, 0)),
            pl.BlockSpec((1, HID), lambda r: (0, 0)),
            pl.BlockSpec((1, HID, HID), lambda r: (r, 0, 0)),
        ],
        out_specs=[
            pl.BlockSpec((N, HID), lambda r: (0, 0)),
            pl.BlockSpec((1, N, HID), lambda r: (r, 0, 0)),
        ],
        out_shape=[
            jax.ShapeDtypeStruct((N, HID), _f32),
            jax.ShapeDtypeStruct((R, N, HID), _f32),
        ],
        scratch_shapes=[pltpu.VMEM((N, HID), _f32)],
    )(part, x, root, b, Wnext)


def _combine_body(p_ref, x_ref, r_ref, b_ref, o_ref, *, relu):
    y = (p_ref[0] + p_ref[1]
         + jnp.dot(x_ref[...], r_ref[...], preferred_element_type=_f32)
         + b_ref[0])
    o_ref[...] = jnp.maximum(y, 0.0) if relu else y


def _combine(part, x, root, b, relu):
    return pl.pallas_call(
        functools.partial(_combine_body, relu=relu),
        grid=(N // _BN,),
        in_specs=[
            pl.BlockSpec((NC, _BN, HID), lambda n: (0, n, 0)),
            pl.BlockSpec((_BN, HID), lambda n: (n, 0)),
            pl.BlockSpec((HID, HID), lambda n: (0, 0)),
            pl.BlockSpec((1, HID), lambda n: (0, 0)),
        ],
        out_specs=pl.BlockSpec((_BN, HID), lambda n: (n, 0)),
        out_shape=jax.ShapeDtypeStruct((N, HID), _f32),
    )(part, x, root, b)


def kernel(edge_index, edge_type, node_emb, W1, root1, b1, W2, root2, b2):
    src = edge_index[0]
    dst = edge_index[1]
    rel = edge_type
    comb = dst * R + rel
    flat_idx = rel * N + src
    comb3 = comb.reshape(NS, CCH, CHUNK)
    comb4 = comb.reshape(NW, NCH, CHUNK)
    flat3 = flat_idx.reshape(NW, NCH, CHUNK)
    dst3 = dst.reshape(NW, NCH, CHUNK)
    zeros_seg = jnp.zeros((SEG,), _f32)
    zeros_nd = jnp.zeros((N, HID), _f32)

    norm3 = _edge_norm(comb3, comb4, zeros_seg)

    xr1 = _per_relation_transform(node_emb, W1).reshape(SEG, HID)
    part1 = _scatter_layer(xr1, flat3, dst3, norm3, zeros_nd)
    x1, xr2 = _combine_transform(part1, node_emb, root1,
                                 b1.reshape(1, HID), W2)
    part2 = _scatter_layer(xr2.reshape(SEG, HID), flat3, dst3, norm3,
                           zeros_nd)
    return _combine(part2, x1, root2, b2.reshape(1, HID), relu=False)
